# Initial kernel scaffold; baseline (speedup 1.0000x reference)
import jax
import jax.numpy as jnp
from jax.experimental import pallas as pl


def kernel(x, edge_index, W1, b1, W2, b2):
    raise NotImplementedError("write your pallas kernel here")


# speedup vs baseline: 7.9111x; 7.9111x over previous
import functools

import jax
import jax.numpy as jnp
from jax import lax
from jax.experimental import pallas as pl
from jax.experimental.pallas import tpu as pltpu
from jax.experimental.pallas import tpu_sc as plsc

N = 10000
NP = 10240
E_EXT = 330000
CHUNK = 128
NS = 16
NC = 2
TPT = 162
E_PAD = NS * TPT * CHUNK
TPT_DEG = E_PAD // (NC * NS * CHUNK)
HALF = 32
DUMP = N
RPT = NP // NS
ALPHA = 0.1
DEGW = 16
K_STEPS = 10
MLP_BLK = 256

_MESH = plsc.VectorSubcoreMesh(core_axis_name="c", subcore_axis_name="s")
_SC_PARAMS = pltpu.CompilerParams(use_tc_tiling_on_sc=False)


def _deg_body(dst_hbm, zeros_hbm, ones_hbm, deg_out, idx_v, ones_v, row_v, degacc):
    cid = lax.axis_index("c")
    sid = lax.axis_index("s")
    pltpu.sync_copy(zeros_hbm, row_v)
    pltpu.sync_copy(row_v, degacc.at[pl.ds(sid * RPT, RPT)])
    pltpu.sync_copy(ones_hbm, ones_v)
    plsc.subcore_barrier()
    base0 = (cid * NS + sid) * TPT_DEG * CHUNK

    @pl.loop(0, TPT_DEG)
    def _(j):
        base = base0 + j * CHUNK
        pltpu.sync_copy(dst_hbm.at[pl.ds(base, CHUNK)], idx_v)
        pltpu.sync_copy(ones_v, degacc.at[idx_v], add=True)

    plsc.subcore_barrier()
    pltpu.sync_copy(degacc.at[pl.ds(sid * RPT, RPT)], row_v)
    pltpu.sync_copy(row_v, deg_out.at[cid, pl.ds(sid * RPT, RPT)])


_deg_call = pl.kernel(
    _deg_body,
    out_type=jax.ShapeDtypeStruct((NC, NP, DEGW), jnp.float32),
    mesh=_MESH,
    scratch_types=[
        pltpu.VMEM((CHUNK,), jnp.int32),
        pltpu.VMEM((CHUNK, DEGW), jnp.float32),
        pltpu.VMEM((RPT, DEGW), jnp.float32),
        pltpu.VMEM_SHARED((NP, DEGW), jnp.float32),
    ],
    compiler_params=_SC_PARAMS,
)


def _prep_body(x_ref, w1_ref, b1_ref, w2_ref, b2_ref, deg_ref,
               g_ref, c_ref, cf_ref, u_ref, uf_ref):
    h1 = jnp.dot(x_ref[...], w1_ref[...], preferred_element_type=jnp.float32)
    h1 = jnp.maximum(h1 + b1_ref[...], 0.0)
    h = jnp.dot(h1, w2_ref[...], preferred_element_type=jnp.float32) + b2_ref[...]
    deg = deg_ref[0, :, 0:1] + deg_ref[1, :, 0:1]
    dis = lax.rsqrt(deg)
    g_ref[...] = h * dis
    c_ref[...] = jnp.broadcast_to((1.0 - ALPHA) * dis * dis, (MLP_BLK, HALF))
    cf_ref[...] = jnp.broadcast_to((1.0 - ALPHA) * dis, (MLP_BLK, HALF))
    u_ref[...] = ALPHA * dis * h
    uf_ref[...] = ALPHA * h


_prep_call = pl.pallas_call(
    _prep_body,
    grid=(NP // MLP_BLK,),
    in_specs=[
        pl.BlockSpec((MLP_BLK, 128), lambda i: (i, 0)),
        pl.BlockSpec((128, 128), lambda i: (0, 0)),
        pl.BlockSpec((1, 128), lambda i: (0, 0)),
        pl.BlockSpec((128, 64), lambda i: (0, 0)),
        pl.BlockSpec((1, 64), lambda i: (0, 0)),
        pl.BlockSpec((NC, MLP_BLK, DEGW), lambda i: (0, i, 0)),
    ],
    out_specs=[
        pl.BlockSpec((MLP_BLK, 64), lambda i: (i, 0)),
        pl.BlockSpec((MLP_BLK, HALF), lambda i: (i, 0)),
        pl.BlockSpec((MLP_BLK, HALF), lambda i: (i, 0)),
        pl.BlockSpec((MLP_BLK, 64), lambda i: (i, 0)),
        pl.BlockSpec((MLP_BLK, 64), lambda i: (i, 0)),
    ],
    out_shape=[
        jax.ShapeDtypeStruct((NP, 64), jnp.float32),
        jax.ShapeDtypeStruct((NP, HALF), jnp.float32),
        jax.ShapeDtypeStruct((NP, HALF), jnp.float32),
        jax.ShapeDtypeStruct((NP, 64), jnp.float32),
        jax.ShapeDtypeStruct((NP, 64), jnp.float32),
    ],
)


def _main_body(gs0, src_hbm, dst_hbm, c_hbm, cf_hbm, us, ufs,
               out_a, out_b,
               isrc, idst, rows, eacc, ec, eu, zbuf, acc, sem):
    cid = lax.axis_index("c")
    sid = lax.axis_index("s")
    row0 = sid * RPT

    @pl.loop(0, RPT)
    def _(i):
        z = jnp.zeros((16,), jnp.float32)
        zbuf[i, pl.ds(0, 16)] = z
        zbuf[i, pl.ds(16, 16)] = z

    ebase = sid * TPT * CHUNK

    def step(gin, gout, c_r, u_r):
        pltpu.sync_copy(zbuf, acc.at[pl.ds(row0, RPT)])
        plsc.subcore_barrier()

        @pl.loop(0, TPT)
        def _(j):
            base = ebase + j * CHUNK
            pltpu.sync_copy(src_hbm.at[pl.ds(base, CHUNK)], isrc)
            pltpu.async_copy(gin.at[isrc], rows, sem).wait()
            pltpu.sync_copy(dst_hbm.at[pl.ds(base, CHUNK)], idst)
            pltpu.sync_copy(rows, acc.at[idst], add=True)

        plsc.subcore_barrier()

        pltpu.sync_copy(acc.at[pl.ds(row0, RPT)], eacc)
        pltpu.sync_copy(c_r.at[pl.ds(row0, RPT)], ec)
        pltpu.sync_copy(u_r.at[pl.ds(row0, RPT)], eu)

        @pl.loop(0, RPT)
        def _(i):
            for c0 in (0, 16):
                a = eacc[i, pl.ds(c0, 16)]
                eacc[i, pl.ds(c0, 16)] = ec[i, pl.ds(c0, 16)] * a + eu[i, pl.ds(c0, 16)]

        pltpu.sync_copy(eacc, gout.at[pl.ds(row0, RPT)])

    g_in = gs0.at[cid]
    buf_a = out_a.at[cid]
    buf_b = out_b.at[cid]
    u_c = us.at[cid]
    uf_c = ufs.at[cid]

    step(g_in, buf_a, c_hbm, u_c)

    @pl.loop(0, (K_STEPS - 2) // 2)
    def _(k):
        step(buf_a, buf_b, c_hbm, u_c)
        step(buf_b, buf_a, c_hbm, u_c)

    step(buf_a, buf_b, cf_hbm, uf_c)


_main_call = pl.kernel(
    _main_body,
    out_type=[
        jax.ShapeDtypeStruct((NC, NP, HALF), jnp.float32),
        jax.ShapeDtypeStruct((NC, NP, HALF), jnp.float32),
    ],
    mesh=_MESH,
    scratch_types=[
        pltpu.VMEM((CHUNK,), jnp.int32),
        pltpu.VMEM((CHUNK,), jnp.int32),
        pltpu.VMEM((CHUNK, HALF), jnp.float32),
        pltpu.VMEM((RPT, HALF), jnp.float32),
        pltpu.VMEM((RPT, HALF), jnp.float32),
        pltpu.VMEM((RPT, HALF), jnp.float32),
        pltpu.VMEM((RPT, HALF), jnp.float32),
        pltpu.VMEM_SHARED((NP, HALF), jnp.float32),
        pltpu.SemaphoreType.DMA,
    ],
    compiler_params=_SC_PARAMS,
)


@jax.jit
def kernel(x, edge_index, W1, b1, W2, b2):
    src = edge_index[0].astype(jnp.int32)
    dst = edge_index[1].astype(jnp.int32)
    loop_idx = jnp.arange(N, dtype=jnp.int32)
    pad_n = E_PAD - E_EXT
    src_p = jnp.concatenate([src, loop_idx, jnp.zeros((pad_n,), jnp.int32)])
    dst_p = jnp.concatenate([dst, loop_idx, jnp.full((pad_n,), DUMP, jnp.int32)])

    deg_partial = _deg_call(
        dst_p,
        jnp.zeros((RPT, DEGW), jnp.float32),
        jnp.ones((CHUNK, DEGW), jnp.float32),
    )

    xp = jnp.pad(x, ((0, NP - N), (0, 0)))
    g0, c_arr, cf_arr, u_arr, uf_arr = _prep_call(
        xp, W1, b1.reshape(1, -1), W2, b2.reshape(1, -1), deg_partial
    )

    gs0 = jnp.stack([g0[:, :HALF], g0[:, HALF:]])
    us = jnp.stack([u_arr[:, :HALF], u_arr[:, HALF:]])
    ufs = jnp.stack([uf_arr[:, :HALF], uf_arr[:, HALF:]])

    _, out_b = _main_call(gs0, src_p, dst_p, c_arr, cf_arr, us, ufs)
    return jnp.concatenate([out_b[0, :N], out_b[1, :N]], axis=1)


# speedup vs baseline: 16.8637x; 2.1316x over previous
import functools

import jax
import jax.numpy as jnp
from jax import lax
from jax.experimental import pallas as pl
from jax.experimental.pallas import tpu as pltpu
from jax.experimental.pallas import tpu_sc as plsc

N = 10000
NP = 10240
E_EXT = 330000
CHUNK = 128
NS = 16
NC = 2
TPT = 164
E_PAD = NS * TPT * CHUNK
GRP = 4
NG = TPT // GRP
TPT_DEG = E_PAD // (NC * NS * CHUNK)
HALF = 32
DUMP = N
RPT = NP // NS
ALPHA = 0.1
DEGW = 16
K_STEPS = 10
MLP_BLK = 256

_MESH = plsc.VectorSubcoreMesh(core_axis_name="c", subcore_axis_name="s")
_SC_PARAMS = pltpu.CompilerParams(use_tc_tiling_on_sc=False)


def _deg_body(dst_hbm, zeros_hbm, ones_hbm, deg_out, idx_v, ones_v, row_v, degacc):
    cid = lax.axis_index("c")
    sid = lax.axis_index("s")
    pltpu.sync_copy(zeros_hbm, row_v)
    pltpu.sync_copy(row_v, degacc.at[pl.ds(sid * RPT, RPT)])
    pltpu.sync_copy(ones_hbm, ones_v)
    plsc.subcore_barrier()
    base0 = (cid * NS + sid) * TPT_DEG * CHUNK

    @pl.loop(0, TPT_DEG)
    def _(j):
        base = base0 + j * CHUNK
        pltpu.sync_copy(dst_hbm.at[pl.ds(base, CHUNK)], idx_v)
        pltpu.sync_copy(ones_v, degacc.at[idx_v], add=True)

    plsc.subcore_barrier()
    pltpu.sync_copy(degacc.at[pl.ds(sid * RPT, RPT)], row_v)
    pltpu.sync_copy(row_v, deg_out.at[cid, pl.ds(sid * RPT, RPT)])


_deg_call = pl.kernel(
    _deg_body,
    out_type=jax.ShapeDtypeStruct((NC, NP, DEGW), jnp.float32),
    mesh=_MESH,
    scratch_types=[
        pltpu.VMEM((CHUNK,), jnp.int32),
        pltpu.VMEM((CHUNK, DEGW), jnp.float32),
        pltpu.VMEM((RPT, DEGW), jnp.float32),
        pltpu.VMEM_SHARED((NP, DEGW), jnp.float32),
    ],
    compiler_params=_SC_PARAMS,
)


def _prep_body(x_ref, w1_ref, b1_ref, w2_ref, b2_ref, deg_ref,
               g_ref, c_ref, cf_ref, u_ref, uf_ref):
    h1 = jnp.dot(x_ref[...], w1_ref[...], preferred_element_type=jnp.float32)
    h1 = jnp.maximum(h1 + b1_ref[...], 0.0)
    h = jnp.dot(h1, w2_ref[...], preferred_element_type=jnp.float32) + b2_ref[...]
    deg = deg_ref[0, :, 0:1] + deg_ref[1, :, 0:1]
    dis = lax.rsqrt(deg)
    g_ref[...] = h * dis
    c_ref[...] = jnp.broadcast_to((1.0 - ALPHA) * dis * dis, (MLP_BLK, HALF))
    cf_ref[...] = jnp.broadcast_to((1.0 - ALPHA) * dis, (MLP_BLK, HALF))
    u_ref[...] = ALPHA * dis * h
    uf_ref[...] = ALPHA * h


_prep_call = pl.pallas_call(
    _prep_body,
    grid=(NP // MLP_BLK,),
    in_specs=[
        pl.BlockSpec((MLP_BLK, 128), lambda i: (i, 0)),
        pl.BlockSpec((128, 128), lambda i: (0, 0)),
        pl.BlockSpec((1, 128), lambda i: (0, 0)),
        pl.BlockSpec((128, 64), lambda i: (0, 0)),
        pl.BlockSpec((1, 64), lambda i: (0, 0)),
        pl.BlockSpec((NC, MLP_BLK, DEGW), lambda i: (0, i, 0)),
    ],
    out_specs=[
        pl.BlockSpec((MLP_BLK, 64), lambda i: (i, 0)),
        pl.BlockSpec((MLP_BLK, HALF), lambda i: (i, 0)),
        pl.BlockSpec((MLP_BLK, HALF), lambda i: (i, 0)),
        pl.BlockSpec((MLP_BLK, 64), lambda i: (i, 0)),
        pl.BlockSpec((MLP_BLK, 64), lambda i: (i, 0)),
    ],
    out_shape=[
        jax.ShapeDtypeStruct((NP, 64), jnp.float32),
        jax.ShapeDtypeStruct((NP, HALF), jnp.float32),
        jax.ShapeDtypeStruct((NP, HALF), jnp.float32),
        jax.ShapeDtypeStruct((NP, 64), jnp.float32),
        jax.ShapeDtypeStruct((NP, 64), jnp.float32),
    ],
)


def _main_body(gs0, src_hbm, dst_hbm, c_hbm, cf_hbm, us, ufs,
               out_a, out_b,
               isrc_all, idst_all, rows_a, rows_b, eacc, ec, eu, zbuf, acc,
               sem_a, sem_b):
    cid = lax.axis_index("c")
    sid = lax.axis_index("s")
    row0 = sid * RPT

    pltpu.sync_copy(src_hbm.at[sid], isrc_all)
    pltpu.sync_copy(dst_hbm.at[sid], idst_all)

    @pl.loop(0, CHUNK)
    def _(i):
        z = jnp.zeros((16,), jnp.float32)
        zbuf[i, pl.ds(0, 16)] = z
        zbuf[i, pl.ds(16, 16)] = z

    def step(gin, gout, c_r, u_r):
        @pl.loop(0, RPT // CHUNK)
        def _(i):
            pltpu.sync_copy(zbuf, acc.at[pl.ds(row0 + i * CHUNK, CHUNK)])

        plsc.subcore_barrier()

        def fire(bank, sem, g):
            for b in range(GRP):
                pltpu.async_copy(gin.at[isrc_all.at[g * GRP + b]], bank.at[b], sem)

        def drain(bank, sem):
            for b in range(GRP):
                pltpu.make_async_copy(gin.at[pl.ds(0, CHUNK)], bank.at[b], sem).wait()

        def scat(bank, g):
            for b in range(GRP):
                pltpu.sync_copy(bank.at[b], acc.at[idst_all.at[g * GRP + b]], add=True)

        fire(rows_a, sem_a, 0)

        @pl.loop(0, (NG - 1) // 2)
        def _(dg):
            ga = 2 * dg
            fire(rows_b, sem_b, ga + 1)
            drain(rows_a, sem_a)
            scat(rows_a, ga)
            fire(rows_a, sem_a, ga + 2)
            drain(rows_b, sem_b)
            scat(rows_b, ga + 1)

        drain(rows_a, sem_a)
        scat(rows_a, NG - 1)

        plsc.subcore_barrier()

        @pl.loop(0, RPT // CHUNK)
        def _(i):
            r = row0 + i * CHUNK
            pltpu.sync_copy(acc.at[pl.ds(r, CHUNK)], eacc)
            pltpu.sync_copy(c_r.at[pl.ds(r, CHUNK)], ec)
            pltpu.sync_copy(u_r.at[pl.ds(r, CHUNK)], eu)

            @pl.loop(0, CHUNK)
            def _(ii):
                for c0 in (0, 16):
                    a = eacc[ii, pl.ds(c0, 16)]
                    eacc[ii, pl.ds(c0, 16)] = ec[ii, pl.ds(c0, 16)] * a + eu[ii, pl.ds(c0, 16)]

            pltpu.sync_copy(eacc, gout.at[pl.ds(r, CHUNK)])

    g_in = gs0.at[cid]
    buf_a = out_a.at[cid]
    buf_b = out_b.at[cid]
    u_c = us.at[cid]
    uf_c = ufs.at[cid]

    step(g_in, buf_a, c_hbm, u_c)

    @pl.loop(0, (K_STEPS - 2) // 2)
    def _(k):
        step(buf_a, buf_b, c_hbm, u_c)
        step(buf_b, buf_a, c_hbm, u_c)

    step(buf_a, buf_b, cf_hbm, uf_c)


_main_call = pl.kernel(
    _main_body,
    out_type=[
        jax.ShapeDtypeStruct((NC, NP, HALF), jnp.float32),
        jax.ShapeDtypeStruct((NC, NP, HALF), jnp.float32),
    ],
    mesh=_MESH,
    scratch_types=[
        pltpu.VMEM((TPT, CHUNK), jnp.int32),
        pltpu.VMEM((TPT, CHUNK), jnp.int32),
        pltpu.VMEM((GRP, CHUNK, HALF), jnp.float32),
        pltpu.VMEM((GRP, CHUNK, HALF), jnp.float32),
        pltpu.VMEM((CHUNK, HALF), jnp.float32),
        pltpu.VMEM((CHUNK, HALF), jnp.float32),
        pltpu.VMEM((CHUNK, HALF), jnp.float32),
        pltpu.VMEM((CHUNK, HALF), jnp.float32),
        pltpu.VMEM_SHARED((NP, HALF), jnp.float32),
        pltpu.SemaphoreType.DMA,
        pltpu.SemaphoreType.DMA,
    ],
    compiler_params=_SC_PARAMS,
)


@jax.jit
def kernel(x, edge_index, W1, b1, W2, b2):
    src = edge_index[0].astype(jnp.int32)
    dst = edge_index[1].astype(jnp.int32)
    loop_idx = jnp.arange(N, dtype=jnp.int32)
    pad_n = E_PAD - E_EXT
    src_p = jnp.concatenate([src, loop_idx, jnp.zeros((pad_n,), jnp.int32)])
    dst_p = jnp.concatenate([dst, loop_idx, jnp.full((pad_n,), DUMP, jnp.int32)])

    deg_partial = _deg_call(
        dst_p,
        jnp.zeros((RPT, DEGW), jnp.float32),
        jnp.ones((CHUNK, DEGW), jnp.float32),
    )

    xp = jnp.pad(x, ((0, NP - N), (0, 0)))
    g0, c_arr, cf_arr, u_arr, uf_arr = _prep_call(
        xp, W1, b1.reshape(1, -1), W2, b2.reshape(1, -1), deg_partial
    )

    gs0 = jnp.stack([g0[:, :HALF], g0[:, HALF:]])
    us = jnp.stack([u_arr[:, :HALF], u_arr[:, HALF:]])
    ufs = jnp.stack([uf_arr[:, :HALF], uf_arr[:, HALF:]])

    src3 = src_p.reshape(NS, TPT, CHUNK)
    dst3 = dst_p.reshape(NS, TPT, CHUNK)
    _, out_b = _main_call(gs0, src3, dst3, c_arr, cf_arr, us, ufs)
    return jnp.concatenate([out_b[0, :N], out_b[1, :N]], axis=1)
